# baseline (speedup 1.0000x reference)
import functools

import jax
import jax.numpy as jnp
from jax import lax
from jax.experimental import pallas as pl
from jax.experimental.pallas import tpu as pltpu
from jax.experimental.pallas import tpu_sc as plsc

_NC = 2
_NS = 16
_NBUF = 4
_CHUNKS = 1


@functools.lru_cache(maxsize=None)
def _make_gather(B, D, CB, off):
    NW = _NC * _NS
    RPW = CB // NW
    NCH = 2 * _NBUF
    CH = RPW // NCH
    mesh = plsc.VectorSubcoreMesh(core_axis_name="c", subcore_axis_name="s")

    @functools.partial(
        pl.kernel,
        out_type=(jax.ShapeDtypeStruct((CB, D), jnp.float32),
                  jax.ShapeDtypeStruct((CB, D), jnp.float32)),
        mesh=mesh,
        scratch_types=(
            [pltpu.VMEM((RPW,), jnp.int32),
             pltpu.VMEM((RPW,), jnp.int32)]
            + [pltpu.VMEM((CH, D), jnp.float32) for _ in range(_NBUF)]
            + [pltpu.SemaphoreType.DMA for _ in range(2 * _NBUF)]
        ),
    )
    def gather_k(name_tab, dom_tab, name_idx, dom_idx, ne_out, de_out,
                 idx_n, idx_d, *bufsem):
        bufs = bufsem[:_NBUF]
        gsem = bufsem[_NBUF:2 * _NBUF]
        wsem = bufsem[2 * _NBUF:]
        wid = lax.axis_index("s") * _NC + lax.axis_index("c")
        base = wid * RPW
        pltpu.sync_copy(name_idx.at[pl.ds(off + base, RPW)], idx_n)
        pltpu.sync_copy(dom_idx.at[pl.ds(off + base, RPW)], idx_d)

        jobs = ([(name_tab, idx_n, ne_out, c) for c in range(NCH)]
                + [(dom_tab, idx_d, de_out, c) for c in range(NCH)])

        def gather(j):
            tab, idx_v, _, c = jobs[j]
            k = j % _NBUF
            return pltpu.async_copy(
                tab.at[idx_v.at[pl.ds(c * CH, CH)]], bufs[k], gsem[k])

        def write(j):
            _, _, out, c = jobs[j]
            k = j % _NBUF
            return pltpu.async_copy(
                bufs[k], out.at[pl.ds(base + c * CH, CH)], wsem[k])

        n_jobs = len(jobs)
        g = [None] * n_jobs
        w = [None] * n_jobs
        for j in range(_NBUF):
            g[j] = gather(j)
        for j in range(n_jobs):
            g[j].wait()
            w[j] = write(j)
            nxt = j + _NBUF
            if nxt < n_jobs:
                w[j].wait()
                g[nxt] = gather(nxt)
        for j in range(n_jobs - _NBUF, n_jobs):
            w[j].wait()

    return gather_k


def _mlp_body(ne_ref, de_ref, w1a_ref, w1b_ref, b1_ref, w2_ref, b2_ref,
              out_ref):
    h = jnp.dot(ne_ref[...].astype(jnp.bfloat16),
                w1a_ref[...].astype(jnp.bfloat16),
                preferred_element_type=jnp.float32)
    h = h + jnp.dot(de_ref[...].astype(jnp.bfloat16),
                    w1b_ref[...].astype(jnp.bfloat16),
                    preferred_element_type=jnp.float32)
    h = jnp.maximum(h + b1_ref[...], 0.0)
    v = jnp.dot(h, w2_ref[...], preferred_element_type=jnp.float32)
    out_ref[...] = 5.0 * jax.nn.sigmoid(v[:, 0] + b2_ref[0])


@functools.lru_cache(maxsize=None)
def _make_mlp(B, D, H, BLK):
    grid = (B // BLK,)
    return pl.pallas_call(
        _mlp_body,
        grid=grid,
        in_specs=[
            pl.BlockSpec((BLK, D), lambda i: (i, 0)),
            pl.BlockSpec((BLK, D), lambda i: (i, 0)),
            pl.BlockSpec((D, H), lambda i: (0, 0)),
            pl.BlockSpec((D, H), lambda i: (1, 0)),
            pl.BlockSpec((1, H), lambda i: (0, 0)),
            pl.BlockSpec((H, 1), lambda i: (0, 0)),
            pl.BlockSpec(memory_space=pltpu.SMEM),
        ],
        out_specs=pl.BlockSpec((BLK,), lambda i: (i,)),
        out_shape=jax.ShapeDtypeStruct((B,), jnp.float32),
    )


def kernel(name_indices, domain_indices, name_table, domain_table,
           W1, b1, W2, b2):
    B = name_indices.shape[0]
    D = name_table.shape[1]
    H = W1.shape[1]
    ni = (name_indices if name_indices.dtype == jnp.int32
          else name_indices.astype(jnp.int32))
    di = (domain_indices if domain_indices.dtype == jnp.int32
          else domain_indices.astype(jnp.int32))
    CB0 = (3 * B) // 4
    b1r = b1.reshape(1, H)
    outs = []
    for cb, off in ((CB0, 0), (B - CB0, CB0)):
        ne, de = _make_gather(B, D, cb, off)(
            name_table, domain_table, ni, di)
        outs.append(_make_mlp(cb, D, H, min(cb, 4096))(
            ne, de, W1, W1, b1r, W2, b2))
    return jnp.concatenate(outs)


# speedup vs baseline: 1.0345x; 1.0345x over previous
import functools

import jax
import jax.numpy as jnp
from jax import lax
from jax.experimental import pallas as pl
from jax.experimental.pallas import tpu as pltpu
from jax.experimental.pallas import tpu_sc as plsc

_NC = 2
_NS = 16
_NBUF = 4
_CHUNKS = 1


@functools.lru_cache(maxsize=None)
def _make_gather(B, D, CB, off):
    NW = _NC * _NS
    RPW = CB // NW
    NCH = 2 * _NBUF
    CH = RPW // NCH
    mesh = plsc.VectorSubcoreMesh(core_axis_name="c", subcore_axis_name="s")

    @functools.partial(
        pl.kernel,
        out_type=(jax.ShapeDtypeStruct((CB, D), jnp.float32),
                  jax.ShapeDtypeStruct((CB, D), jnp.float32)),
        mesh=mesh,
        scratch_types=(
            [pltpu.VMEM((RPW,), jnp.int32),
             pltpu.VMEM((RPW,), jnp.int32)]
            + [pltpu.VMEM((CH, D), jnp.float32) for _ in range(_NBUF)]
            + [pltpu.SemaphoreType.DMA for _ in range(2 * _NBUF)]
        ),
    )
    def gather_k(name_tab, dom_tab, name_idx, dom_idx, ne_out, de_out,
                 idx_n, idx_d, *bufsem):
        bufs = bufsem[:_NBUF]
        gsem = bufsem[_NBUF:2 * _NBUF]
        wsem = bufsem[2 * _NBUF:]
        wid = lax.axis_index("s") * _NC + lax.axis_index("c")
        base = wid * RPW
        pltpu.sync_copy(name_idx.at[pl.ds(off + base, RPW)], idx_n)
        pltpu.sync_copy(dom_idx.at[pl.ds(off + base, RPW)], idx_d)

        jobs = ([(name_tab, idx_n, ne_out, c) for c in range(NCH)]
                + [(dom_tab, idx_d, de_out, c) for c in range(NCH)])

        def gather(j):
            tab, idx_v, _, c = jobs[j]
            k = j % _NBUF
            return pltpu.async_copy(
                tab.at[idx_v.at[pl.ds(c * CH, CH)]], bufs[k], gsem[k])

        def write(j):
            _, _, out, c = jobs[j]
            k = j % _NBUF
            return pltpu.async_copy(
                bufs[k], out.at[pl.ds(base + c * CH, CH)], wsem[k])

        n_jobs = len(jobs)
        g = [None] * n_jobs
        w = [None] * n_jobs
        for j in range(_NBUF):
            g[j] = gather(j)
        for j in range(n_jobs):
            g[j].wait()
            w[j] = write(j)
            nxt = j + _NBUF
            if nxt < n_jobs:
                w[j].wait()
                g[nxt] = gather(nxt)
        for j in range(n_jobs - _NBUF, n_jobs):
            w[j].wait()

    return gather_k


def _mlp_body(ne_ref, de_ref, w1a_ref, w1b_ref, b1_ref, w2_ref, b2_ref,
              out_ref):
    h = jnp.dot(ne_ref[...].astype(jnp.bfloat16),
                w1a_ref[...].astype(jnp.bfloat16),
                preferred_element_type=jnp.float32)
    h = h + jnp.dot(de_ref[...].astype(jnp.bfloat16),
                    w1b_ref[...].astype(jnp.bfloat16),
                    preferred_element_type=jnp.float32)
    h = jnp.maximum(h + b1_ref[...], 0.0)
    v = jnp.dot(h, w2_ref[...], preferred_element_type=jnp.float32)
    out_ref[...] = 5.0 * jax.nn.sigmoid(v[:, 0] + b2_ref[0])


@functools.lru_cache(maxsize=None)
def _make_mlp(B, D, H, BLK):
    grid = (B // BLK,)
    return pl.pallas_call(
        _mlp_body,
        grid=grid,
        in_specs=[
            pl.BlockSpec((BLK, D), lambda i: (i, 0)),
            pl.BlockSpec((BLK, D), lambda i: (i, 0)),
            pl.BlockSpec((D, H), lambda i: (0, 0)),
            pl.BlockSpec((D, H), lambda i: (1, 0)),
            pl.BlockSpec((1, H), lambda i: (0, 0)),
            pl.BlockSpec((H, 1), lambda i: (0, 0)),
            pl.BlockSpec(memory_space=pltpu.SMEM),
        ],
        out_specs=pl.BlockSpec((BLK,), lambda i: (i,)),
        out_shape=jax.ShapeDtypeStruct((B,), jnp.float32),
    )


def kernel(name_indices, domain_indices, name_table, domain_table,
           W1, b1, W2, b2):
    B = name_indices.shape[0]
    D = name_table.shape[1]
    H = W1.shape[1]
    ni = (name_indices if name_indices.dtype == jnp.int32
          else name_indices.astype(jnp.int32))
    di = (domain_indices if domain_indices.dtype == jnp.int32
          else domain_indices.astype(jnp.int32))
    b1r = b1.reshape(1, H)
    ne, de = _make_gather(B, D, B, 0)(name_table, domain_table, ni, di)
    return _make_mlp(B, D, H, 4096)(ne, de, W1, W1, b1r, W2, b2)


# speedup vs baseline: 1.0660x; 1.0304x over previous
import functools

import jax
import jax.numpy as jnp
from jax import lax
from jax.experimental import pallas as pl
from jax.experimental.pallas import tpu as pltpu
from jax.experimental.pallas import tpu_sc as plsc

_NC = 2
_NS = 16
_NBUF = 4
_CHUNKS = 1


@functools.lru_cache(maxsize=None)
def _make_gather(B, D, CB, off):
    NW = _NC * _NS
    RPW = CB // NW
    NCH = 2 * _NBUF
    CH = RPW // NCH
    mesh = plsc.VectorSubcoreMesh(core_axis_name="c", subcore_axis_name="s")

    @functools.partial(
        pl.kernel,
        out_type=(jax.ShapeDtypeStruct((CB, D), jnp.float32),
                  jax.ShapeDtypeStruct((CB, D), jnp.float32)),
        mesh=mesh,
        scratch_types=(
            [pltpu.VMEM((RPW,), jnp.int32),
             pltpu.VMEM((RPW,), jnp.int32)]
            + [pltpu.VMEM((CH, D), jnp.float32) for _ in range(_NBUF)]
            + [pltpu.SemaphoreType.DMA for _ in range(2 * _NBUF)]
        ),
    )
    def gather_k(name_tab, dom_tab, name_idx, dom_idx, ne_out, de_out,
                 idx_n, idx_d, *bufsem):
        bufs = bufsem[:_NBUF]
        gsem = bufsem[_NBUF:2 * _NBUF]
        wsem = bufsem[2 * _NBUF:]
        wid = lax.axis_index("s") * _NC + lax.axis_index("c")
        base = wid * RPW
        pltpu.sync_copy(name_idx.at[pl.ds(off + base, RPW)], idx_n)
        pltpu.sync_copy(dom_idx.at[pl.ds(off + base, RPW)], idx_d)

        jobs = []
        for c in range(NCH):
            jobs.append((name_tab, idx_n, ne_out, c))
            jobs.append((dom_tab, idx_d, de_out, c))

        def gather(j):
            tab, idx_v, _, c = jobs[j]
            k = j % _NBUF
            return pltpu.async_copy(
                tab.at[idx_v.at[pl.ds(c * CH, CH)]], bufs[k], gsem[k])

        def write(j):
            _, _, out, c = jobs[j]
            k = j % _NBUF
            return pltpu.async_copy(
                bufs[k], out.at[pl.ds(base + c * CH, CH)], wsem[k])

        n_jobs = len(jobs)
        g = [None] * n_jobs
        w = [None] * n_jobs
        for j in range(_NBUF):
            g[j] = gather(j)
        for j in range(n_jobs):
            g[j].wait()
            w[j] = write(j)
            nxt = j + _NBUF
            if nxt < n_jobs:
                w[j].wait()
                g[nxt] = gather(nxt)
        for j in range(n_jobs - _NBUF, n_jobs):
            w[j].wait()

    return gather_k


def _mlp_body(ne_ref, de_ref, w1a_ref, w1b_ref, b1_ref, w2_ref, b2_ref,
              out_ref):
    h = jnp.dot(ne_ref[...].astype(jnp.bfloat16),
                w1a_ref[...].astype(jnp.bfloat16),
                preferred_element_type=jnp.float32)
    h = h + jnp.dot(de_ref[...].astype(jnp.bfloat16),
                    w1b_ref[...].astype(jnp.bfloat16),
                    preferred_element_type=jnp.float32)
    h = jnp.maximum(h + b1_ref[...], 0.0)
    v = jnp.dot(h, w2_ref[...], preferred_element_type=jnp.float32)
    out_ref[...] = 5.0 * jax.nn.sigmoid(v[:, 0] + b2_ref[0])


@functools.lru_cache(maxsize=None)
def _make_mlp(B, D, H, BLK):
    grid = (B // BLK,)
    return pl.pallas_call(
        _mlp_body,
        grid=grid,
        in_specs=[
            pl.BlockSpec((BLK, D), lambda i: (i, 0)),
            pl.BlockSpec((BLK, D), lambda i: (i, 0)),
            pl.BlockSpec((D, H), lambda i: (0, 0)),
            pl.BlockSpec((D, H), lambda i: (1, 0)),
            pl.BlockSpec((1, H), lambda i: (0, 0)),
            pl.BlockSpec((H, 1), lambda i: (0, 0)),
            pl.BlockSpec(memory_space=pltpu.SMEM),
        ],
        out_specs=pl.BlockSpec((BLK,), lambda i: (i,)),
        out_shape=jax.ShapeDtypeStruct((B,), jnp.float32),
    )


def kernel(name_indices, domain_indices, name_table, domain_table,
           W1, b1, W2, b2):
    B = name_indices.shape[0]
    D = name_table.shape[1]
    H = W1.shape[1]
    ni = (name_indices if name_indices.dtype == jnp.int32
          else name_indices.astype(jnp.int32))
    di = (domain_indices if domain_indices.dtype == jnp.int32
          else domain_indices.astype(jnp.int32))
    b1r = b1.reshape(1, H)
    ne, de = _make_gather(B, D, B, 0)(name_table, domain_table, ni, di)
    return _make_mlp(B, D, H, 4096)(ne, de, W1, W1, b1r, W2, b2)


# speedup vs baseline: 1.0689x; 1.0027x over previous
import functools

import jax
import jax.numpy as jnp
from jax import lax
from jax.experimental import pallas as pl
from jax.experimental.pallas import tpu as pltpu
from jax.experimental.pallas import tpu_sc as plsc

_NC = 2
_NS = 16
_NBUF = 4


@functools.lru_cache(maxsize=None)
def _make_gather(B, D, CB, off):
    NW = _NC * _NS
    RPW = CB // NW
    NCH = 2 * _NBUF
    CH = RPW // NCH
    mesh = plsc.VectorSubcoreMesh(core_axis_name="c", subcore_axis_name="s")

    @functools.partial(
        pl.kernel,
        out_type=(jax.ShapeDtypeStruct((CB, D), jnp.float32),
                  jax.ShapeDtypeStruct((CB, D), jnp.float32)),
        mesh=mesh,
        scratch_types=(
            [pltpu.VMEM((RPW,), jnp.int32),
             pltpu.VMEM((RPW,), jnp.int32)]
            + [pltpu.VMEM((CH, D), jnp.float32) for _ in range(_NBUF)]
            + [pltpu.SemaphoreType.DMA for _ in range(2 * _NBUF)]
        ),
    )
    def gather_k(name_tab, dom_tab, name_idx, dom_idx, ne_out, de_out,
                 idx_n, idx_d, *bufsem):
        bufs = bufsem[:_NBUF]
        gsem = bufsem[_NBUF:2 * _NBUF]
        wsem = bufsem[2 * _NBUF:]
        wid = lax.axis_index("s") * _NC + lax.axis_index("c")
        base = wid * RPW
        pltpu.sync_copy(name_idx.at[pl.ds(off + base, RPW)], idx_n)
        pltpu.sync_copy(dom_idx.at[pl.ds(off + base, RPW)], idx_d)

        jobs = []
        for c in range(NCH):
            jobs.append((name_tab, idx_n, ne_out, c))
            jobs.append((dom_tab, idx_d, de_out, c))

        def gather(j):
            tab, idx_v, _, c = jobs[j]
            k = j % _NBUF
            return pltpu.async_copy(
                tab.at[idx_v.at[pl.ds(c * CH, CH)]], bufs[k], gsem[k])

        def write(j):
            _, _, out, c = jobs[j]
            k = j % _NBUF
            return pltpu.async_copy(
                bufs[k], out.at[pl.ds(base + c * CH, CH)], wsem[k])

        n_jobs = len(jobs)
        g = [None] * n_jobs
        w = [None] * n_jobs
        for j in range(_NBUF):
            g[j] = gather(j)
        for j in range(n_jobs):
            g[j].wait()
            w[j] = write(j)
            nxt = j + _NBUF
            if nxt < n_jobs:
                w[j].wait()
                g[nxt] = gather(nxt)
        for j in range(n_jobs - _NBUF, n_jobs):
            w[j].wait()

    return gather_k


def _mlp_body(ne_ref, de_ref, w1a_ref, w1b_ref, b1_ref, w2_ref, b2_ref,
              out_ref):
    h = jnp.dot(ne_ref[...].astype(jnp.bfloat16),
                w1a_ref[...].astype(jnp.bfloat16),
                preferred_element_type=jnp.float32)
    h = h + jnp.dot(de_ref[...].astype(jnp.bfloat16),
                    w1b_ref[...].astype(jnp.bfloat16),
                    preferred_element_type=jnp.float32)
    h = jnp.maximum(h + b1_ref[...], 0.0)
    v = jnp.dot(h, w2_ref[...], preferred_element_type=jnp.float32)
    out_ref[...] = 5.0 * jax.nn.sigmoid(v[:, 0] + b2_ref[0])


@functools.lru_cache(maxsize=None)
def _make_mlp(B, D, H, BLK):
    grid = (B // BLK,)
    return pl.pallas_call(
        _mlp_body,
        grid=grid,
        in_specs=[
            pl.BlockSpec((BLK, D), lambda i: (i, 0)),
            pl.BlockSpec((BLK, D), lambda i: (i, 0)),
            pl.BlockSpec((D, H), lambda i: (0, 0)),
            pl.BlockSpec((D, H), lambda i: (1, 0)),
            pl.BlockSpec((1, H), lambda i: (0, 0)),
            pl.BlockSpec((H, 1), lambda i: (0, 0)),
            pl.BlockSpec(memory_space=pltpu.SMEM),
        ],
        out_specs=pl.BlockSpec((BLK,), lambda i: (i,)),
        out_shape=jax.ShapeDtypeStruct((B,), jnp.float32),
    )


def kernel(name_indices, domain_indices, name_table, domain_table,
           W1, b1, W2, b2):
    B = name_indices.shape[0]
    D = name_table.shape[1]
    H = W1.shape[1]
    ni = (name_indices if name_indices.dtype == jnp.int32
          else name_indices.astype(jnp.int32))
    di = (domain_indices if domain_indices.dtype == jnp.int32
          else domain_indices.astype(jnp.int32))
    b1r = b1.reshape(1, H)
    ne, de = _make_gather(B, D, B, 0)(name_table, domain_table, ni, di)
    return _make_mlp(B, D, H, 4096)(ne, de, W1, W1, b1r, W2, b2)
